# baseline (speedup 1.0000x reference)
import functools

import jax
import jax.numpy as jnp
from jax import lax
from jax.experimental import pallas as pl
from jax.experimental.pallas import tpu as pltpu
from jax.experimental.pallas import tpu_sc as plsc

VOCAB = 1000000
EMBED = 64
SCALE = 8.0

NC = 2
NS = 16
NW = NC * NS

B = 4096 * 200
B_PER_W = B // NW
CHUNK = 800
N_CHUNKS = B_PER_W // CHUNK


def _emb_body(idx_hbm, table_hbm, out_hbm, idx_v, rows_v, sem):
    wid = lax.axis_index("s") * NC + lax.axis_index("c")
    base = wid * B_PER_W

    def chunk_body(i, carry):
        cbase = base + i * CHUNK
        pltpu.sync_copy(idx_hbm.at[pl.ds(cbase, CHUNK)], idx_v)
        pltpu.async_copy(table_hbm.at[idx_v], rows_v, sem).wait()

        def row_body(r, c):
            for j in range(EMBED // 16):
                rows_v[r, pl.ds(j * 16, 16)] = rows_v[r, pl.ds(j * 16, 16)] * SCALE
            return c

        lax.fori_loop(0, CHUNK, row_body, 0)
        pltpu.sync_copy(rows_v, out_hbm.at[pl.ds(cbase, CHUNK)])
        return carry

    lax.fori_loop(0, N_CHUNKS, chunk_body, 0)


@jax.jit
def _embedding(x_flat, table):
    mesh = plsc.VectorSubcoreMesh(core_axis_name="c", subcore_axis_name="s")
    k = functools.partial(
        pl.kernel,
        out_type=jax.ShapeDtypeStruct((B, EMBED), jnp.float32),
        mesh=mesh,
        scratch_types=[
            pltpu.VMEM((CHUNK,), jnp.int32),
            pltpu.VMEM((CHUNK, EMBED), jnp.float32),
            pltpu.SemaphoreType.DMA,
        ],
        compiler_params=pltpu.CompilerParams(use_tc_tiling_on_sc=False),
    )(_emb_body)
    return k(x_flat, table)


def kernel(x, table):
    out = _embedding(x.reshape(-1), table)
    return out.reshape(x.shape + (EMBED,))
